# Initial kernel scaffold; baseline (speedup 1.0000x reference)
import jax
import jax.numpy as jnp
from jax.experimental import pallas as pl


def kernel(x, edge_index, sim_weight, rep, node_signal, W, W_self):
    raise NotImplementedError("write your pallas kernel here")


# speedup vs baseline: 22.7658x; 22.7658x over previous
import jax
import jax.numpy as jnp
from jax import lax
from jax.experimental import pallas as pl
from jax.experimental.pallas import tpu as pltpu
from jax.experimental.pallas import tpu_sc as plsc

_NC = 2
_NS = 16
_NW = _NC * _NS
_L = 16

_N = 10000
_E = 320000
_D = 128
_C = 80
_CPM = 5
_MACRO = 25
_EPW = _E // _NW
_HR = 80
_NA = _N + _HR
_RPT = 840
_NT_IO = _NA // _RPT

_BN = 2000


def _tc_pre_body(x_ref, w_ref, ws_ref, rep_ref, ns_ref, hs_ref, sf_ref):
    xb = x_ref[...]
    dn = (((1,), (1,)), ((), ()))
    hs_ref[...] = ns_ref[...] * lax.dot_general(
        xb, w_ref[...], dn, preferred_element_type=jnp.float32)
    g = 1.0 / (1.0 + jnp.exp(-rep_ref[...]))
    sf_ref[...] = g * lax.dot_general(xb, ws_ref[...], dn,
                                      preferred_element_type=jnp.float32)


def _tc_comb_body(p_ref, d0_ref, d1_ref, sf_ref, o_ref):
    p = p_ref[0] + p_ref[1]
    deg = d0_ref[...] + d1_ref[...]
    o = p / (deg + 1e-6) + sf_ref[...]
    o_ref[...] = jnp.where(o >= 0, o, 0.01 * o)


def _sc_edge_body(hs_hbm, row_hbm, col_hbm, sim_hbm, rep_hbm,
                  zrow_hbm, hidx_hbm,
                  pout_hbm,
                  acc_sh,
                  rowm_v, colm_v, cm_v, rows_v, hist_v, hidx_v, rep_v,
                  sem):
    cid = lax.axis_index("c")
    sid = lax.axis_index("s")
    wid = cid * _NS + sid

    r0 = sid * _RPT

    @pl.when(sid < _NT_IO)
    def _init():
        pltpu.sync_copy(zrow_hbm.at[pl.ds(r0, _RPT)],
                        acc_sh.at[pl.ds(r0, _RPT)])

    pltpu.sync_copy(rep_hbm, rep_v)
    pltpu.sync_copy(hidx_hbm, hidx_v)

    def zhist(r, carry):
        for f in range(_D // _L):
            hist_v[r, pl.ds(f * _L, _L)] = jnp.zeros((_L,), jnp.float32)
        return carry
    lax.fori_loop(0, _HR, zhist, 0)
    plsc.subcore_barrier()

    ones16 = jnp.ones((_L,), jnp.float32)

    def macro(m, carry):
        pltpu.sync_copy(row_hbm.at[wid, m], rowm_v)
        pltpu.sync_copy(col_hbm.at[wid, m], colm_v)
        pltpu.sync_copy(sim_hbm.at[wid, m], cm_v)
        for j in range(_CPM):
            gather = pltpu.async_copy(hs_hbm.at[colm_v.at[j]], rows_v, sem)
            for g in range(_C // _L):
                sl = pl.ds(g * _L, _L)
                riv = rowm_v[j, sl]
                rr = plsc.load_gather(rep_v, [riv])
                rc = plsc.load_gather(rep_v, [colm_v[j, sl]])
                gate = 1.0 / (1.0 + jnp.exp(-(rr + rc)))
                cm_v[j, sl] = cm_v[j, sl] * gate
                plsc.addupdate_scatter(
                    hist_v,
                    [lax.shift_right_logical(riv, 7),
                     lax.bitwise_and(riv, 127)],
                    ones16)
            gather.wait()

            def scale(gi, c2):
                base = gi * _L
                cvec = cm_v[j, pl.ds(base, _L)]
                for jj in range(_L):
                    cb = jnp.full((_L,), cvec[jj], jnp.float32)
                    for f in range(_D // _L):
                        fs = pl.ds(f * _L, _L)
                        rows_v[base + jj, fs] = rows_v[base + jj, fs] * cb
                return c2
            lax.fori_loop(0, _C // _L, scale, 0)

            pltpu.sync_copy(rows_v, acc_sh.at[rowm_v.at[j]], add=True)
        return carry

    lax.fori_loop(0, _MACRO, macro, 0)

    pltpu.sync_copy(hist_v, acc_sh.at[hidx_v], add=True)
    plsc.subcore_barrier()

    @pl.when(sid < _NT_IO)
    def _writeback():
        pltpu.sync_copy(acc_sh.at[pl.ds(r0, _RPT)],
                        pout_hbm.at[cid, pl.ds(r0, _RPT)])


def kernel(x, edge_index, sim_weight, rep, node_signal, W, W_self):
    x = x.astype(jnp.float32)
    rep = rep.astype(jnp.float32)
    ns = node_signal.astype(jnp.float32)
    row = edge_index[0].astype(jnp.int32).reshape(_NW, _MACRO, _CPM, _C)
    col = edge_index[1].astype(jnp.int32).reshape(_NW, _MACRO, _CPM, _C)
    sim = sim_weight.astype(jnp.float32).reshape(_NW, _MACRO, _CPM, _C)

    hs, sf = pl.pallas_call(
        _tc_pre_body,
        grid=(_N // _BN,),
        in_specs=[
            pl.BlockSpec((_BN, _D), lambda i: (i, 0)),
            pl.BlockSpec((_D, _D), lambda i: (0, 0)),
            pl.BlockSpec((_D, _D), lambda i: (0, 0)),
            pl.BlockSpec((_BN, 1), lambda i: (i, 0)),
            pl.BlockSpec((_BN, 1), lambda i: (i, 0)),
        ],
        out_specs=[pl.BlockSpec((_BN, _D), lambda i: (i, 0))] * 2,
        out_shape=[jax.ShapeDtypeStruct((_N, _D), jnp.float32)] * 2,
    )(x, W, W_self, rep.reshape(_N, 1), ns.reshape(_N, 1))

    zrow = jnp.zeros((_NA, _D), jnp.float32)
    hidx = _N + jnp.arange(_HR, dtype=jnp.int32)

    mesh = plsc.VectorSubcoreMesh(core_axis_name="c", subcore_axis_name="s")
    sc = pl.kernel(
        _sc_edge_body,
        out_type=jax.ShapeDtypeStruct((_NC, _NA, _D), jnp.float32),
        mesh=mesh,
        scratch_types=[
            pltpu.VMEM_SHARED((_NA, _D), jnp.float32),
            pltpu.VMEM((_CPM, _C), jnp.int32),
            pltpu.VMEM((_CPM, _C), jnp.int32),
            pltpu.VMEM((_CPM, _C), jnp.float32),
            pltpu.VMEM((_C, _D), jnp.float32),
            pltpu.VMEM((_HR, _D), jnp.float32),
            pltpu.VMEM((_HR,), jnp.int32),
            pltpu.VMEM((_N,), jnp.float32),
            pltpu.SemaphoreType.DMA,
        ],
        compiler_params=pltpu.CompilerParams(needs_layout_passes=False),
    )
    pout = sc(hs, row, col, sim, rep, zrow, hidx)

    pm = pout[:, :_N, :]
    dflat0 = pout[0, _N:, :].reshape(_HR * _D, 1)[:_N]
    dflat1 = pout[1, _N:, :].reshape(_HR * _D, 1)[:_N]

    out = pl.pallas_call(
        _tc_comb_body,
        grid=(_N // _BN,),
        in_specs=[
            pl.BlockSpec((_NC, _BN, _D), lambda i: (0, i, 0)),
            pl.BlockSpec((_BN, 1), lambda i: (i, 0)),
            pl.BlockSpec((_BN, 1), lambda i: (i, 0)),
            pl.BlockSpec((_BN, _D), lambda i: (i, 0)),
        ],
        out_specs=pl.BlockSpec((_BN, _D), lambda i: (i, 0)),
        out_shape=jax.ShapeDtypeStruct((_N, _D), jnp.float32),
    )(pm, dflat0, dflat1, sf)
    return out


# speedup vs baseline: 31.6598x; 1.3907x over previous
import jax
import jax.numpy as jnp
from jax import lax
from jax.experimental import pallas as pl
from jax.experimental.pallas import tpu as pltpu
from jax.experimental.pallas import tpu_sc as plsc

_NC = 2
_NS = 16
_NW = _NC * _NS
_L = 16

_N = 10000
_E = 320000
_D = 128
_C = 80
_CPM = 5
_MACRO = 25
_EPW = _E // _NW
_HR = 80
_NA = _N + _HR
_RPT = 840
_NT_IO = _NA // _RPT

_BN = 2000


def _tc_pre_body(x_ref, w_ref, ws_ref, rep_ref, ns_ref, hs_ref, sf_ref):
    xb = x_ref[...]
    dn = (((1,), (1,)), ((), ()))
    hs_ref[...] = ns_ref[...] * lax.dot_general(
        xb, w_ref[...], dn, preferred_element_type=jnp.float32)
    g = 1.0 / (1.0 + jnp.exp(-rep_ref[...]))
    sf_ref[...] = g * lax.dot_general(xb, ws_ref[...], dn,
                                      preferred_element_type=jnp.float32)


def _tc_comb_body(p_ref, d0_ref, d1_ref, sf_ref, o_ref):
    p = p_ref[0] + p_ref[1]
    deg = d0_ref[...] + d1_ref[...]
    o = p / (deg + 1e-6) + sf_ref[...]
    o_ref[...] = jnp.where(o >= 0, o, 0.01 * o)


def _sc_edge_body(hs_hbm, row_hbm, col_hbm, sim_hbm, rep_hbm,
                  zrow_hbm, hidx_hbm,
                  pout_hbm,
                  acc_sh,
                  rowm_v, colm_v, cm_v, rows0_v, rows1_v, hist_v, hidx_v,
                  rep_v,
                  sg0, sg1, ss0, ss1, sst):
    cid = lax.axis_index("c")
    sid = lax.axis_index("s")
    wid = cid * _NS + sid
    rows = (rows0_v, rows1_v)
    sg = (sg0, sg1)
    ss = (ss0, ss1)
    _SC_BYTES = _C * _D * 4

    def wait_scatter(b):
        pltpu.make_async_copy(rows[b], acc_sh.at[rowm_v.at[0]], ss[b]).wait()

    def wait_gather(b, j):
        pltpu.make_async_copy(hs_hbm.at[colm_v.at[j]], rows[b], sg[b]).wait()

    r0 = sid * _RPT

    @pl.when(sid < _NT_IO)
    def _init():
        pltpu.sync_copy(zrow_hbm.at[pl.ds(r0, _RPT)],
                        acc_sh.at[pl.ds(r0, _RPT)])

    pltpu.sync_copy(rep_hbm, rep_v)
    pltpu.sync_copy(hidx_hbm, hidx_v)

    def zhist(r, carry):
        for f in range(_D // _L):
            fs = pl.ds(f * _L, _L)
            hist_v[r, fs] = jnp.zeros((_L,), jnp.float32)
            rows0_v[r, fs] = jnp.zeros((_L,), jnp.float32)
            rows1_v[r, fs] = jnp.zeros((_L,), jnp.float32)
        return carry
    lax.fori_loop(0, _HR, zhist, 0)
    plsc.subcore_barrier()

    ones16 = jnp.ones((_L,), jnp.float32)

    del _SC_BYTES
    pltpu.async_copy(rows0_v, acc_sh.at[hidx_v], ss0, add=True)
    pltpu.async_copy(rows1_v, acc_sh.at[hidx_v], ss1, add=True)

    def macro(m, carry):
        wait_scatter(1)
        wait_scatter(0)
        pltpu.async_copy(row_hbm.at[wid, m], rowm_v, sst)
        pltpu.async_copy(col_hbm.at[wid, m], colm_v, sst)
        pltpu.async_copy(sim_hbm.at[wid, m], cm_v, sst)
        pltpu.make_async_copy(row_hbm.at[wid, m], rowm_v, sst).wait()
        pltpu.make_async_copy(col_hbm.at[wid, m], colm_v, sst).wait()
        pltpu.make_async_copy(sim_hbm.at[wid, m], cm_v, sst).wait()
        pltpu.async_copy(hs_hbm.at[colm_v.at[0]], rows0_v, sg0)
        for j in range(_CPM):
            b = j % 2
            for g in range(_C // _L):
                sl = pl.ds(g * _L, _L)
                riv = rowm_v[j, sl]
                rr = plsc.load_gather(rep_v, [riv])
                rc = plsc.load_gather(rep_v, [colm_v[j, sl]])
                gate = 1.0 / (1.0 + jnp.exp(-(rr + rc)))
                cm_v[j, sl] = cm_v[j, sl] * gate
                plsc.addupdate_scatter(
                    hist_v,
                    [lax.shift_right_logical(riv, 7),
                     lax.bitwise_and(riv, 127)],
                    ones16)
            if j < _CPM - 1:
                if j >= 1:
                    wait_scatter(1 - b)
                pltpu.async_copy(hs_hbm.at[colm_v.at[j + 1]], rows[1 - b],
                                 sg[1 - b])
            wait_gather(b, j)

            def scale(gi, c2):
                base = gi * _L
                cvec = cm_v[j, pl.ds(base, _L)]
                for jj in range(_L):
                    cb = jnp.full((_L,), cvec[jj], jnp.float32)
                    for f in range(_D // _L):
                        fs = pl.ds(f * _L, _L)
                        rows[b][base + jj, fs] = rows[b][base + jj, fs] * cb
                return c2
            lax.fori_loop(0, _C // _L, scale, 0)

            pltpu.async_copy(rows[b], acc_sh.at[rowm_v.at[j]], ss[b],
                             add=True)
        return carry

    lax.fori_loop(0, _MACRO, macro, 0)
    wait_scatter(1)
    wait_scatter(0)

    pltpu.sync_copy(hist_v, acc_sh.at[hidx_v], add=True)
    plsc.subcore_barrier()

    @pl.when(sid < _NT_IO)
    def _writeback():
        pltpu.sync_copy(acc_sh.at[pl.ds(r0, _RPT)],
                        pout_hbm.at[cid, pl.ds(r0, _RPT)])


def kernel(x, edge_index, sim_weight, rep, node_signal, W, W_self):
    x = x.astype(jnp.float32)
    rep = rep.astype(jnp.float32)
    ns = node_signal.astype(jnp.float32)
    row = edge_index[0].astype(jnp.int32).reshape(_NW, _MACRO, _CPM, _C)
    col = edge_index[1].astype(jnp.int32).reshape(_NW, _MACRO, _CPM, _C)
    sim = sim_weight.astype(jnp.float32).reshape(_NW, _MACRO, _CPM, _C)

    hs, sf = pl.pallas_call(
        _tc_pre_body,
        grid=(_N // _BN,),
        in_specs=[
            pl.BlockSpec((_BN, _D), lambda i: (i, 0)),
            pl.BlockSpec((_D, _D), lambda i: (0, 0)),
            pl.BlockSpec((_D, _D), lambda i: (0, 0)),
            pl.BlockSpec((_BN, 1), lambda i: (i, 0)),
            pl.BlockSpec((_BN, 1), lambda i: (i, 0)),
        ],
        out_specs=[pl.BlockSpec((_BN, _D), lambda i: (i, 0))] * 2,
        out_shape=[jax.ShapeDtypeStruct((_N, _D), jnp.float32)] * 2,
    )(x, W, W_self, rep.reshape(_N, 1), ns.reshape(_N, 1))

    zrow = jnp.zeros((_NA, _D), jnp.float32)
    hidx = _N + jnp.arange(_HR, dtype=jnp.int32)

    mesh = plsc.VectorSubcoreMesh(core_axis_name="c", subcore_axis_name="s")
    sc = pl.kernel(
        _sc_edge_body,
        out_type=jax.ShapeDtypeStruct((_NC, _NA, _D), jnp.float32),
        mesh=mesh,
        scratch_types=[
            pltpu.VMEM_SHARED((_NA, _D), jnp.float32),
            pltpu.VMEM((_CPM, _C), jnp.int32),
            pltpu.VMEM((_CPM, _C), jnp.int32),
            pltpu.VMEM((_CPM, _C), jnp.float32),
            pltpu.VMEM((_C, _D), jnp.float32),
            pltpu.VMEM((_C, _D), jnp.float32),
            pltpu.VMEM((_HR, _D), jnp.float32),
            pltpu.VMEM((_HR,), jnp.int32),
            pltpu.VMEM((_N,), jnp.float32),
            pltpu.SemaphoreType.DMA,
            pltpu.SemaphoreType.DMA,
            pltpu.SemaphoreType.DMA,
            pltpu.SemaphoreType.DMA,
            pltpu.SemaphoreType.DMA,
        ],
        compiler_params=pltpu.CompilerParams(needs_layout_passes=False),
    )
    pout = sc(hs, row, col, sim, rep, zrow, hidx)

    pm = pout[:, :_N, :]
    dflat0 = pout[0, _N:, :].reshape(_HR * _D, 1)[:_N]
    dflat1 = pout[1, _N:, :].reshape(_HR * _D, 1)[:_N]

    out = pl.pallas_call(
        _tc_comb_body,
        grid=(_N // _BN,),
        in_specs=[
            pl.BlockSpec((_NC, _BN, _D), lambda i: (0, i, 0)),
            pl.BlockSpec((_BN, 1), lambda i: (i, 0)),
            pl.BlockSpec((_BN, 1), lambda i: (i, 0)),
            pl.BlockSpec((_BN, _D), lambda i: (i, 0)),
        ],
        out_specs=pl.BlockSpec((_BN, _D), lambda i: (i, 0)),
        out_shape=jax.ShapeDtypeStruct((_N, _D), jnp.float32),
    )(pm, dflat0, dflat1, sf)
    return out


# speedup vs baseline: 32.4884x; 1.0262x over previous
import jax
import jax.numpy as jnp
from jax import lax
from jax.experimental import pallas as pl
from jax.experimental.pallas import tpu as pltpu
from jax.experimental.pallas import tpu_sc as plsc

_NC = 2
_NS = 16
_NW = _NC * _NS
_L = 16

_N = 10000
_E = 320000
_D = 128
_C = 80
_CPM = 5
_MACRO = 25
_EPW = _E // _NW
_HR = 80
_NA = _N + _HR
_RPT = 840
_NT_IO = _NA // _RPT

_BN = 2000


def _tc_pre_body(x_ref, w_ref, ws_ref, rep_ref, ns_ref, hs_ref, sf_ref):
    xb = x_ref[...]
    dn = (((1,), (1,)), ((), ()))
    hs_ref[...] = ns_ref[...] * lax.dot_general(
        xb, w_ref[...], dn, preferred_element_type=jnp.float32)
    g = 1.0 / (1.0 + jnp.exp(-rep_ref[...]))
    sf_ref[...] = g * lax.dot_general(xb, ws_ref[...], dn,
                                      preferred_element_type=jnp.float32)


def _tc_comb_body(p_ref, d0_ref, d1_ref, sf_ref, o_ref):
    p = p_ref[0] + p_ref[1]
    deg = d0_ref[...] + d1_ref[...]
    o = p / (deg + 1e-6) + sf_ref[...]
    o_ref[...] = jnp.where(o >= 0, o, 0.01 * o)


def _sc_edge_body(hs_hbm, row_hbm, col_hbm, sim_hbm, rep_hbm,
                  zrow_hbm, hidx_hbm,
                  pout_hbm,
                  acc_sh,
                  rowm_v, colm_v, cm_v, rows0_v, rows1_v, hist_v, hidx_v,
                  rep_v,
                  sg0, sg1, ss0, ss1, sst):
    cid = lax.axis_index("c")
    sid = lax.axis_index("s")
    wid = cid * _NS + sid
    rows = (rows0_v, rows1_v)
    sg = (sg0, sg1)
    ss = (ss0, ss1)
    _SC_BYTES = _C * _D * 4

    def wait_scatter(b):
        pltpu.make_async_copy(rows[b], acc_sh.at[rowm_v.at[0]], ss[b]).wait()

    def wait_gather(b, j):
        pltpu.make_async_copy(hs_hbm.at[colm_v.at[j]], rows[b], sg[b]).wait()

    r0 = sid * _RPT

    @pl.when(sid < _NT_IO)
    def _init():
        pltpu.sync_copy(zrow_hbm.at[pl.ds(r0, _RPT)],
                        acc_sh.at[pl.ds(r0, _RPT)])

    pltpu.sync_copy(rep_hbm, rep_v)
    pltpu.sync_copy(hidx_hbm, hidx_v)

    def zhist(r, carry):
        for f in range(_D // _L):
            fs = pl.ds(f * _L, _L)
            hist_v[r, fs] = jnp.zeros((_L,), jnp.float32)
            rows0_v[r, fs] = jnp.zeros((_L,), jnp.float32)
            rows1_v[r, fs] = jnp.zeros((_L,), jnp.float32)
        return carry
    lax.fori_loop(0, _HR, zhist, 0)
    plsc.subcore_barrier()

    ones16 = jnp.ones((_L,), jnp.float32)

    del _SC_BYTES
    pltpu.async_copy(rows0_v, acc_sh.at[hidx_v], ss0, add=True)
    pltpu.async_copy(rows1_v, acc_sh.at[hidx_v], ss1, add=True)

    def macro(m, carry):
        wait_scatter(1)
        wait_scatter(0)
        pltpu.async_copy(row_hbm.at[wid, m], rowm_v, sst)
        pltpu.async_copy(col_hbm.at[wid, m], colm_v, sst)
        pltpu.async_copy(sim_hbm.at[wid, m], cm_v, sst)
        pltpu.make_async_copy(row_hbm.at[wid, m], rowm_v, sst).wait()
        pltpu.make_async_copy(col_hbm.at[wid, m], colm_v, sst).wait()
        pltpu.make_async_copy(sim_hbm.at[wid, m], cm_v, sst).wait()
        pltpu.async_copy(hs_hbm.at[colm_v.at[0]], rows0_v, sg0)
        for j in range(_CPM):
            b = j % 2
            for g in range(_C // _L):
                sl = pl.ds(g * _L, _L)
                riv = rowm_v[j, sl]
                rr = plsc.load_gather(rep_v, [riv])
                rc = plsc.load_gather(rep_v, [colm_v[j, sl]])
                gate = 1.0 / (1.0 + jnp.exp(-(rr + rc)))
                cm_v[j, sl] = cm_v[j, sl] * gate
                plsc.addupdate_scatter(
                    hist_v,
                    [lax.shift_right_logical(riv, 7),
                     lax.bitwise_and(riv, 127)],
                    ones16)
            if j < _CPM - 1:
                if j >= 1:
                    wait_scatter(1 - b)
                pltpu.async_copy(hs_hbm.at[colm_v.at[j + 1]], rows[1 - b],
                                 sg[1 - b])
            wait_gather(b, j)

            def scale(gi, c2):
                base = gi * _L
                cvec = cm_v[j, pl.ds(base, _L)]
                for jj in range(_L):
                    cb = jnp.full((_L,), cvec[jj], jnp.float32)
                    for f in range(_D // _L):
                        fs = pl.ds(f * _L, _L)
                        rows[b][base + jj, fs] = rows[b][base + jj, fs] * cb
                return c2
            lax.fori_loop(0, _C // _L, scale, 0)

            pltpu.async_copy(rows[b], acc_sh.at[rowm_v.at[j]], ss[b],
                             add=True)
        return carry

    lax.fori_loop(0, _MACRO, macro, 0)
    wait_scatter(1)
    wait_scatter(0)

    pltpu.sync_copy(hist_v, acc_sh.at[hidx_v], add=True)
    plsc.subcore_barrier()

    @pl.when(sid < _NT_IO)
    def _writeback():
        pltpu.sync_copy(acc_sh.at[pl.ds(r0, _RPT)],
                        pout_hbm.at[cid, pl.ds(r0, _RPT)])


def kernel(x, edge_index, sim_weight, rep, node_signal, W, W_self):
    x = x.astype(jnp.float32)
    rep = rep.astype(jnp.float32)
    ns = node_signal.astype(jnp.float32)
    row = edge_index[0].astype(jnp.int32).reshape(_NW, _MACRO, _CPM, _C)
    col = edge_index[1].astype(jnp.int32).reshape(_NW, _MACRO, _CPM, _C)
    sim = sim_weight.astype(jnp.float32).reshape(_NW, _MACRO, _CPM, _C)

    hs, sf = pl.pallas_call(
        _tc_pre_body,
        grid=(_N // _BN,),
        in_specs=[
            pl.BlockSpec((_BN, _D), lambda i: (i, 0)),
            pl.BlockSpec((_D, _D), lambda i: (0, 0)),
            pl.BlockSpec((_D, _D), lambda i: (0, 0)),
            pl.BlockSpec((_BN, 1), lambda i: (i, 0)),
            pl.BlockSpec((_BN, 1), lambda i: (i, 0)),
        ],
        out_specs=[pl.BlockSpec((_BN, _D), lambda i: (i, 0))] * 2,
        out_shape=[jax.ShapeDtypeStruct((_N, _D), jnp.float32)] * 2,
    )(x, W, W_self, rep.reshape(_N, 1), ns.reshape(_N, 1))

    zrow = jnp.zeros((_NA, _D), jnp.float32)
    hidx = _N + jnp.arange(_HR, dtype=jnp.int32)

    mesh = plsc.VectorSubcoreMesh(core_axis_name="c", subcore_axis_name="s")
    sc = pl.kernel(
        _sc_edge_body,
        out_type=jax.ShapeDtypeStruct((_NC, _NA, _D), jnp.float32),
        mesh=mesh,
        scratch_types=[
            pltpu.VMEM_SHARED((_NA, _D), jnp.float32),
            pltpu.VMEM((_CPM, _C), jnp.int32),
            pltpu.VMEM((_CPM, _C), jnp.int32),
            pltpu.VMEM((_CPM, _C), jnp.float32),
            pltpu.VMEM((_C, _D), jnp.float32),
            pltpu.VMEM((_C, _D), jnp.float32),
            pltpu.VMEM((_HR, _D), jnp.float32),
            pltpu.VMEM((_HR,), jnp.int32),
            pltpu.VMEM((_N,), jnp.float32),
            pltpu.SemaphoreType.DMA,
            pltpu.SemaphoreType.DMA,
            pltpu.SemaphoreType.DMA,
            pltpu.SemaphoreType.DMA,
            pltpu.SemaphoreType.DMA,
        ],
        compiler_params=pltpu.CompilerParams(needs_layout_passes=False),
    )
    pout = sc(hs, row, col, sim, rep, zrow, hidx)

    dflat0 = pout[0, _N:, :].reshape(_HR * _D, 1)[:_N]
    dflat1 = pout[1, _N:, :].reshape(_HR * _D, 1)[:_N]

    out = pl.pallas_call(
        _tc_comb_body,
        grid=(_N // _BN,),
        in_specs=[
            pl.BlockSpec((_NC, _BN, _D), lambda i: (0, i, 0)),
            pl.BlockSpec((_BN, 1), lambda i: (i, 0)),
            pl.BlockSpec((_BN, 1), lambda i: (i, 0)),
            pl.BlockSpec((_BN, _D), lambda i: (i, 0)),
        ],
        out_specs=pl.BlockSpec((_BN, _D), lambda i: (i, 0)),
        out_shape=jax.ShapeDtypeStruct((_N, _D), jnp.float32),
    )(pout, dflat0, dflat1, sf)
    return out


# speedup vs baseline: 32.8307x; 1.0105x over previous
import jax
import jax.numpy as jnp
from jax import lax
from jax.experimental import pallas as pl
from jax.experimental.pallas import tpu as pltpu
from jax.experimental.pallas import tpu_sc as plsc

_NC = 2
_NS = 16
_NW = _NC * _NS
_L = 16

_N = 10000
_E = 320000
_D = 128
_C = 64
_CPM = 5
_MACRO = 32
_EPW = _C * _CPM * _MACRO
_NM = 10048
_HR = 79
_NA = 10160
_RPT = 1016
_NT_IO = _NA // _RPT
_BUF = (0, 1, 2, 0, 1)

_BN = 2000


def _tc_pre_body(x_ref, w_ref, ws_ref, rep_ref, ns_ref, hs_ref, sf_ref):
    xb = x_ref[...]
    dn = (((1,), (1,)), ((), ()))
    hs_ref[...] = ns_ref[...] * lax.dot_general(
        xb, w_ref[...], dn, preferred_element_type=jnp.float32)
    g = 1.0 / (1.0 + jnp.exp(-rep_ref[...]))
    sf_ref[...] = g * lax.dot_general(xb, ws_ref[...], dn,
                                      preferred_element_type=jnp.float32)


def _tc_comb_body(p_ref, d0_ref, d1_ref, sf_ref, o_ref):
    p = p_ref[0] + p_ref[1]
    deg = d0_ref[...] + d1_ref[...]
    o = p / (deg + 1e-6) + sf_ref[...]
    o_ref[...] = jnp.where(o >= 0, o, 0.01 * o)


def _sc_edge_body(hs_hbm, row_hbm, col_hbm, sim_hbm, rep_hbm,
                  zrow_hbm, hidx_hbm,
                  pout_hbm,
                  acc_sh,
                  rowm_v, colm_v, cm_v, rows0_v, rows1_v, rows2_v,
                  sidx_v, hist_v, hidx_v, rep_v,
                  sg0, sg1, sg2, ss0, ss1, ss2, sst):
    cid = lax.axis_index("c")
    sid = lax.axis_index("s")
    wid = cid * _NS + sid
    rows = (rows0_v, rows1_v, rows2_v)
    sg = (sg0, sg1, sg2)
    ss = (ss0, ss1, ss2)

    def wait_scatter(b):
        pltpu.make_async_copy(rows[b], acc_sh.at[sidx_v.at[b]], ss[b]).wait()

    def wait_gather(b, j):
        pltpu.make_async_copy(hs_hbm.at[colm_v.at[j]], rows[b], sg[b]).wait()

    r0 = sid * _RPT

    @pl.when(sid < _NT_IO)
    def _init():
        pltpu.sync_copy(zrow_hbm.at[pl.ds(r0, _RPT)],
                        acc_sh.at[pl.ds(r0, _RPT)])

    pltpu.sync_copy(rep_hbm, rep_v)
    pltpu.sync_copy(hidx_hbm, hidx_v)

    def zhist(r, carry):
        for f in range(_D // _L):
            fs = pl.ds(f * _L, _L)
            hist_v[r, fs] = jnp.zeros((_L,), jnp.float32)
            rows0_v[r, fs] = jnp.zeros((_L,), jnp.float32)
            rows1_v[r, fs] = jnp.zeros((_L,), jnp.float32)
            rows2_v[r, fs] = jnp.zeros((_L,), jnp.float32)
        return carry
    lax.fori_loop(0, _C, zhist, 0)

    def zhist2(r, carry):
        for f in range(_D // _L):
            hist_v[_C + r, pl.ds(f * _L, _L)] = jnp.zeros((_L,), jnp.float32)
        return carry
    lax.fori_loop(0, _HR - _C, zhist2, 0)
    for g in range(_C // _L):
        sidx_v[0, pl.ds(g * _L, _L)] = hidx_v[pl.ds(0, _L)]
        sidx_v[1, pl.ds(g * _L, _L)] = hidx_v[pl.ds(0, _L)]
        sidx_v[2, pl.ds(g * _L, _L)] = hidx_v[pl.ds(0, _L)]
    plsc.subcore_barrier()

    ones16 = jnp.ones((_L,), jnp.float32)

    pltpu.async_copy(rows0_v, acc_sh.at[sidx_v.at[0]], ss0, add=True)
    pltpu.async_copy(rows1_v, acc_sh.at[sidx_v.at[1]], ss1, add=True)
    pltpu.async_copy(rows2_v, acc_sh.at[sidx_v.at[2]], ss2, add=True)

    def macro(m, carry):
        pltpu.async_copy(row_hbm.at[wid, m], rowm_v, sst)
        pltpu.async_copy(col_hbm.at[wid, m], colm_v, sst)
        pltpu.async_copy(sim_hbm.at[wid, m], cm_v, sst)
        pltpu.make_async_copy(row_hbm.at[wid, m], rowm_v, sst).wait()
        pltpu.make_async_copy(col_hbm.at[wid, m], colm_v, sst).wait()
        pltpu.make_async_copy(sim_hbm.at[wid, m], cm_v, sst).wait()
        wait_scatter(0)
        pltpu.async_copy(hs_hbm.at[colm_v.at[0]], rows0_v, sg0)
        for j in range(_CPM):
            b = _BUF[j]
            for g in range(_C // _L):
                sl = pl.ds(g * _L, _L)
                riv = rowm_v[j, sl]
                sidx_v[b, sl] = riv
                rr = plsc.load_gather(rep_v, [riv])
                rc = plsc.load_gather(rep_v, [colm_v[j, sl]])
                gate = 1.0 / (1.0 + jnp.exp(-(rr + rc)))
                cm_v[j, sl] = cm_v[j, sl] * gate
                plsc.addupdate_scatter(
                    hist_v,
                    [lax.shift_right_logical(riv, 7),
                     lax.bitwise_and(riv, 127)],
                    ones16)
            if j < _CPM - 1:
                wait_scatter(_BUF[j + 1])
                pltpu.async_copy(hs_hbm.at[colm_v.at[j + 1]], rows[_BUF[j + 1]],
                                 sg[_BUF[j + 1]])
            wait_gather(b, j)

            def scale(gi, c2):
                base = gi * _L
                cvec = cm_v[j, pl.ds(base, _L)]
                for jj in range(_L):
                    cb = jnp.full((_L,), cvec[jj], jnp.float32)
                    for f in range(_D // _L):
                        fs = pl.ds(f * _L, _L)
                        rows[b][base + jj, fs] = rows[b][base + jj, fs] * cb
                return c2
            lax.fori_loop(0, _C // _L, scale, 0)

            pltpu.async_copy(rows[b], acc_sh.at[sidx_v.at[b]], ss[b],
                             add=True)
        return carry

    lax.fori_loop(0, _MACRO, macro, 0)
    wait_scatter(0)
    wait_scatter(1)
    wait_scatter(2)

    pltpu.sync_copy(hist_v, acc_sh.at[hidx_v], add=True)
    plsc.subcore_barrier()

    @pl.when(sid < _NT_IO)
    def _writeback():
        pltpu.sync_copy(acc_sh.at[pl.ds(r0, _RPT)],
                        pout_hbm.at[cid, pl.ds(r0, _RPT)])


def kernel(x, edge_index, sim_weight, rep, node_signal, W, W_self):
    x = x.astype(jnp.float32)
    rep = rep.astype(jnp.float32)
    ns = node_signal.astype(jnp.float32)
    pad = _NW * _EPW - _E
    ppos = jnp.arange(pad, dtype=jnp.int32)
    row = jnp.concatenate(
        [edge_index[0].astype(jnp.int32), _N + ppos % (_NM - _N)]
    ).reshape(_NW, _MACRO, _CPM, _C)
    col = jnp.concatenate(
        [edge_index[1].astype(jnp.int32), ppos % _N]
    ).reshape(_NW, _MACRO, _CPM, _C)
    sim = jnp.concatenate(
        [sim_weight.astype(jnp.float32), jnp.zeros((pad,), jnp.float32)]
    ).reshape(_NW, _MACRO, _CPM, _C)

    hs, sf = pl.pallas_call(
        _tc_pre_body,
        grid=(_N // _BN,),
        in_specs=[
            pl.BlockSpec((_BN, _D), lambda i: (i, 0)),
            pl.BlockSpec((_D, _D), lambda i: (0, 0)),
            pl.BlockSpec((_D, _D), lambda i: (0, 0)),
            pl.BlockSpec((_BN, 1), lambda i: (i, 0)),
            pl.BlockSpec((_BN, 1), lambda i: (i, 0)),
        ],
        out_specs=[pl.BlockSpec((_BN, _D), lambda i: (i, 0))] * 2,
        out_shape=[jax.ShapeDtypeStruct((_N, _D), jnp.float32)] * 2,
    )(x, W, W_self, rep.reshape(_N, 1), ns.reshape(_N, 1))

    zrow = jnp.zeros((_NA, _D), jnp.float32)
    hidx = _NM + jnp.arange(_HR, dtype=jnp.int32)

    mesh = plsc.VectorSubcoreMesh(core_axis_name="c", subcore_axis_name="s")
    sc = pl.kernel(
        _sc_edge_body,
        out_type=jax.ShapeDtypeStruct((_NC, _NA, _D), jnp.float32),
        mesh=mesh,
        scratch_types=[
            pltpu.VMEM_SHARED((_NA, _D), jnp.float32),
            pltpu.VMEM((_CPM, _C), jnp.int32),
            pltpu.VMEM((_CPM, _C), jnp.int32),
            pltpu.VMEM((_CPM, _C), jnp.float32),
            pltpu.VMEM((_C, _D), jnp.float32),
            pltpu.VMEM((_C, _D), jnp.float32),
            pltpu.VMEM((_C, _D), jnp.float32),
            pltpu.VMEM((3, _C), jnp.int32),
            pltpu.VMEM((_HR, _D), jnp.float32),
            pltpu.VMEM((_HR,), jnp.int32),
            pltpu.VMEM((_N,), jnp.float32),
            pltpu.SemaphoreType.DMA,
            pltpu.SemaphoreType.DMA,
            pltpu.SemaphoreType.DMA,
            pltpu.SemaphoreType.DMA,
            pltpu.SemaphoreType.DMA,
            pltpu.SemaphoreType.DMA,
            pltpu.SemaphoreType.DMA,
        ],
        compiler_params=pltpu.CompilerParams(needs_layout_passes=False),
    )
    pout = sc(hs, row, col, sim, rep, zrow, hidx)

    dflat0 = pout[0, _NM:_NM + _HR, :].reshape(_HR * _D, 1)[:_N]
    dflat1 = pout[1, _NM:_NM + _HR, :].reshape(_HR * _D, 1)[:_N]

    out = pl.pallas_call(
        _tc_comb_body,
        grid=(_N // _BN,),
        in_specs=[
            pl.BlockSpec((_NC, _BN, _D), lambda i: (0, i, 0)),
            pl.BlockSpec((_BN, 1), lambda i: (i, 0)),
            pl.BlockSpec((_BN, 1), lambda i: (i, 0)),
            pl.BlockSpec((_BN, _D), lambda i: (i, 0)),
        ],
        out_specs=pl.BlockSpec((_BN, _D), lambda i: (i, 0)),
        out_shape=jax.ShapeDtypeStruct((_N, _D), jnp.float32),
    )(pout, dflat0, dflat1, sf)
    return out
